# Initial kernel scaffold; baseline (speedup 1.0000x reference)
import jax
import jax.numpy as jnp
from jax.experimental import pallas as pl


def kernel(logits, targets):
    raise NotImplementedError("write your pallas kernel here")


# speedup vs baseline: 26.5912x; 26.5912x over previous
import functools

import jax
import jax.numpy as jnp
import numpy as np
from jax import lax
from jax.experimental import pallas as pl
from jax.experimental.pallas import tpu as pltpu
from jax.experimental.pallas import tpu_sc as plsc

NB = 32768
SHIFT = 16
NPIX = 512 * 512
C = 19
BATCH = 4
UNITS = C * BATCH
NWORKERS = 32
CHUNK = 16384
NCHUNK = NPIX // CHUNK
ROWS = 256
LANES = 128


def _dv_table() -> np.ndarray:
    b = np.arange(NB, dtype=np.uint64)
    pat = np.minimum((b << SHIFT) + (1 << (SHIFT - 1)), 0x7F7FFFFF)
    v = pat.astype(np.uint32).view(np.float32).astype(np.float64)
    dv = np.diff(np.concatenate([[0.0], v]))
    return dv.reshape(ROWS, LANES).astype(np.float32)


_DV = _dv_table()


def _sc_hist_body(logits_ref, labels_ref, out_n_ref, out_g_ref,
                  nhist, ghist, logbuf, lblbuf):
    wid = lax.axis_index("s") * 2 + lax.axis_index("c")

    def run_unit(u):
        c = u // BATCH
        b = u % BATCH

        def zero_body(i, _):
            z = jnp.zeros((16,), jnp.float32)
            nhist[pl.ds(i * 16, 16)] = z
            ghist[pl.ds(i * 16, 16)] = z
            return 0

        lax.fori_loop(0, NB // 16, zero_body, 0)

        def chunk_body(k, _):
            off = k * CHUNK
            pltpu.sync_copy(logits_ref.at[b, c, pl.ds(off, CHUNK)], logbuf)
            pltpu.sync_copy(labels_ref.at[b, pl.ds(off, CHUNK)], lblbuf)

            def vreg_body(i, _):
                lbl = lblbuf[pl.ds(i * 16, 16)]
                p = logbuf[pl.ds(i * 16, 16)]
                fg = lbl == c
                e = jnp.abs(jnp.where(fg, 1.0 - p, p))
                key = lax.bitcast_convert_type(e, jnp.int32) >> SHIFT
                cnt_n, last_n = plsc.scan_count(key)
                plsc.addupdate_scatter(nhist, [key],
                                       cnt_n.astype(jnp.float32), mask=last_n)
                cnt_g, last_g = plsc.scan_count(key, mask=fg)
                plsc.addupdate_scatter(ghist, [key],
                                       cnt_g.astype(jnp.float32), mask=last_g)
                return 0

            lax.fori_loop(0, CHUNK // 16, vreg_body, 0)
            return 0

        lax.fori_loop(0, NCHUNK, chunk_body, 0)
        pltpu.sync_copy(nhist, out_n_ref.at[c, b])
        pltpu.sync_copy(ghist, out_g_ref.at[c, b])

    for k in range(3):
        u = wid + k * NWORKERS
        if k * NWORKERS + NWORKERS <= UNITS:
            run_unit(u)
        else:
            @pl.when(u < UNITS)
            def _():
                run_unit(u)


def _sc_histograms(logits_flat, labels_flat):
    mesh = plsc.VectorSubcoreMesh(core_axis_name="c", subcore_axis_name="s",
                                  num_cores=2, num_subcores=16)
    kern = pl.kernel(
        _sc_hist_body,
        out_type=(
            jax.ShapeDtypeStruct((C, BATCH, NB), jnp.float32),
            jax.ShapeDtypeStruct((C, BATCH, NB), jnp.float32),
        ),
        mesh=mesh,
        compiler_params=pltpu.CompilerParams(needs_layout_passes=False),
        scratch_types=[
            pltpu.VMEM((NB,), jnp.float32),
            pltpu.VMEM((NB,), jnp.float32),
            pltpu.VMEM((CHUNK,), jnp.float32),
            pltpu.VMEM((CHUNK,), jnp.int32),
        ],
    )
    return kern(logits_flat, labels_flat)


def _tc_body(nh_ref, gh_ref, dv_ref, out_ref, acc_ref):
    c = pl.program_id(0)

    n = jnp.sum(nh_ref[0], axis=0)
    g = jnp.sum(gh_ref[0], axis=0)

    ntot = jnp.sum(n)
    gtot = jnp.sum(g)

    li = lax.broadcasted_iota(jnp.int32, (LANES, LANES), 0)
    lj = lax.broadcasted_iota(jnp.int32, (LANES, LANES), 1)
    lower_incl = (li >= lj).astype(jnp.float32)
    ri = lax.broadcasted_iota(jnp.int32, (ROWS, ROWS), 0)
    rj = lax.broadcasted_iota(jnp.int32, (ROWS, ROWS), 1)
    ustrict = (rj > ri).astype(jnp.float32)

    def suffix_incl(x):
        row_suf = jnp.dot(x, lower_incl, preferred_element_type=jnp.float32,
                          precision=lax.Precision.HIGHEST)
        row_tot = row_suf[:, 0:1]
        row_off = jnp.dot(ustrict, row_tot, preferred_element_type=jnp.float32,
                          precision=lax.Precision.HIGHEST)
        return row_suf + row_off

    n_ge = suffix_incl(n)
    g_ge = suffix_incl(g)
    denom = gtot + n_ge - g_ge
    j = jnp.where(n_ge > 0, n_ge / jnp.maximum(denom, 1e-30), 0.0)
    loss_c = jnp.sum(dv_ref[...] * j)
    present = (gtot > 0).astype(jnp.float32)

    @pl.when(c == 0)
    def _():
        acc_ref[0] = 0.0
        acc_ref[1] = 0.0

    acc_ref[0] += loss_c * present
    acc_ref[1] += present

    @pl.when(c == C - 1)
    def _():
        out_ref[0, 0] = acc_ref[0] / jnp.maximum(acc_ref[1], 1.0)


def _tc_reduce(nh, gh):
    return pl.pallas_call(
        _tc_body,
        grid=(C,),
        in_specs=[
            pl.BlockSpec((1, BATCH, ROWS, LANES), lambda c: (c, 0, 0, 0)),
            pl.BlockSpec((1, BATCH, ROWS, LANES), lambda c: (c, 0, 0, 0)),
            pl.BlockSpec((ROWS, LANES), lambda c: (0, 0)),
        ],
        out_specs=pl.BlockSpec(memory_space=pltpu.SMEM),
        out_shape=jax.ShapeDtypeStruct((1, 1), jnp.float32),
        scratch_shapes=[pltpu.SMEM((2,), jnp.float32)],
    )(nh, gh, jnp.asarray(_DV))


def kernel(logits, targets):
    logits_flat = logits.reshape(BATCH, C, NPIX)
    labels_flat = targets.reshape(BATCH, NPIX).astype(jnp.int32)
    nh, gh = _sc_histograms(logits_flat, labels_flat)
    out = _tc_reduce(nh.reshape(C, BATCH, ROWS, LANES),
                     gh.reshape(C, BATCH, ROWS, LANES))
    return out.reshape(())


# speedup vs baseline: 59.8282x; 2.2499x over previous
import functools

import jax
import jax.numpy as jnp
import numpy as np
from jax import lax
from jax.experimental import pallas as pl
from jax.experimental.pallas import tpu as pltpu
from jax.experimental.pallas import tpu_sc as plsc

NB = 32768
SHIFT = 16
NPIX = 512 * 512
C = 19
BATCH = 4
UNITS = C * BATCH
NWORKERS = 32
CHUNK = 8192
NCHUNK = NPIX // CHUNK
UNROLL = 4
ROWS = 256
LANES = 128


def _dv_table() -> np.ndarray:
    b = np.arange(NB, dtype=np.uint64)
    pat = np.minimum((b << SHIFT) + (1 << (SHIFT - 1)), 0x7F7FFFFF)
    v = pat.astype(np.uint32).view(np.float32).astype(np.float64)
    dv = np.diff(np.concatenate([[0.0], v]))
    return dv.reshape(ROWS, LANES).astype(np.float32)


_DV = _dv_table()


def _sc_hist_body(logits_ref, labels_ref, out_n_ref, out_g_ref,
                  nhist, ghist, logbuf, lblbuf, seml, semt):
    wid = lax.axis_index("s") * 2 + lax.axis_index("c")

    def run_unit(u):
        c = u // BATCH
        b = u % BATCH

        def zero_body(i, _):
            z = jnp.zeros((16,), jnp.float32)
            nhist[pl.ds(i * 16, 16)] = z
            ghist[pl.ds(i * 16, 16)] = z
            return 0

        lax.fori_loop(0, NB // 16, zero_body, 0)

        def copies(k):
            s = k % 2
            off = k * CHUNK
            return (
                pltpu.make_async_copy(
                    logits_ref.at[b, c, pl.ds(off, CHUNK)], logbuf.at[s],
                    seml.at[s]),
                pltpu.make_async_copy(
                    labels_ref.at[b, pl.ds(off, CHUNK)], lblbuf.at[s],
                    semt.at[s]),
            )

        def process_vreg(lbl, p):
            fg = lbl == c
            e = jnp.abs(jnp.where(fg, 1.0 - p, p))
            key = lax.bitcast_convert_type(e, jnp.int32) >> SHIFT
            return key, fg

        for cp in copies(0):
            cp.start()

        def chunk_body(k, _):
            s = k % 2

            @pl.when(k + 1 < NCHUNK)
            def _():
                for cp in copies(k + 1):
                    cp.start()

            for cp in copies(k):
                cp.wait()

            def vreg_body(i, _):
                base = i * (16 * UNROLL)
                kfs = []
                for uu in range(UNROLL):
                    lbl = lblbuf[s, pl.ds(base + uu * 16, 16)]
                    p = logbuf[s, pl.ds(base + uu * 16, 16)]
                    kfs.append(process_vreg(lbl, p))
                scans = []
                for key, fg in kfs:
                    scans.append(plsc.scan_count(key))
                    scans.append(plsc.scan_count(key, mask=fg))
                for ui, (key, fg) in enumerate(kfs):
                    cnt_n, last_n = scans[2 * ui]
                    cnt_g, last_g = scans[2 * ui + 1]
                    plsc.addupdate_scatter(nhist, [key],
                                           cnt_n.astype(jnp.float32),
                                           mask=last_n)
                    plsc.addupdate_scatter(ghist, [key],
                                           cnt_g.astype(jnp.float32),
                                           mask=last_g)
                return 0

            lax.fori_loop(0, CHUNK // (16 * UNROLL), vreg_body, 0)
            return 0

        lax.fori_loop(0, NCHUNK, chunk_body, 0)

        pltpu.sync_copy(nhist, out_n_ref.at[c, b])
        pltpu.sync_copy(ghist, out_g_ref.at[c, b])

    for k in range(3):
        u = wid + k * NWORKERS
        if k * NWORKERS + NWORKERS <= UNITS:
            run_unit(u)
        else:
            @pl.when(u < UNITS)
            def _():
                run_unit(u)


def _sc_histograms(logits_flat, labels_flat):
    mesh = plsc.VectorSubcoreMesh(core_axis_name="c", subcore_axis_name="s",
                                  num_cores=2, num_subcores=16)
    kern = pl.kernel(
        _sc_hist_body,
        out_type=(
            jax.ShapeDtypeStruct((C, BATCH, NB), jnp.float32),
            jax.ShapeDtypeStruct((C, BATCH, NB), jnp.float32),
        ),
        mesh=mesh,
        compiler_params=pltpu.CompilerParams(needs_layout_passes=False),
        scratch_types=[
            pltpu.VMEM((NB,), jnp.float32),
            pltpu.VMEM((NB,), jnp.float32),
            pltpu.VMEM((2, CHUNK), jnp.float32),
            pltpu.VMEM((2, CHUNK), jnp.int32),
            pltpu.SemaphoreType.DMA((2,)),
            pltpu.SemaphoreType.DMA((2,)),
        ],
    )
    return kern(logits_flat, labels_flat)


def _tc_body(nh_ref, gh_ref, dv_ref, out_ref, acc_ref):
    c = pl.program_id(0)

    n = jnp.sum(nh_ref[0], axis=0)
    g = jnp.sum(gh_ref[0], axis=0)

    ntot = jnp.sum(n)
    gtot = jnp.sum(g)

    li = lax.broadcasted_iota(jnp.int32, (LANES, LANES), 0)
    lj = lax.broadcasted_iota(jnp.int32, (LANES, LANES), 1)
    lower_incl = (li >= lj).astype(jnp.float32)
    ri = lax.broadcasted_iota(jnp.int32, (ROWS, ROWS), 0)
    rj = lax.broadcasted_iota(jnp.int32, (ROWS, ROWS), 1)
    ustrict = (rj > ri).astype(jnp.float32)

    def suffix_incl(x):
        row_suf = jnp.dot(x, lower_incl, preferred_element_type=jnp.float32,
                          precision=lax.Precision.HIGHEST)
        row_tot = row_suf[:, 0:1]
        row_off = jnp.dot(ustrict, row_tot, preferred_element_type=jnp.float32,
                          precision=lax.Precision.HIGHEST)
        return row_suf + row_off

    n_ge = suffix_incl(n)
    g_ge = suffix_incl(g)
    denom = gtot + n_ge - g_ge
    j = jnp.where(n_ge > 0, n_ge / jnp.maximum(denom, 1e-30), 0.0)
    loss_c = jnp.sum(dv_ref[...] * j)
    present = (gtot > 0).astype(jnp.float32)

    @pl.when(c == 0)
    def _():
        acc_ref[0] = 0.0
        acc_ref[1] = 0.0

    acc_ref[0] += loss_c * present
    acc_ref[1] += present

    @pl.when(c == C - 1)
    def _():
        out_ref[0, 0] = acc_ref[0] / jnp.maximum(acc_ref[1], 1.0)


def _tc_reduce(nh, gh):
    return pl.pallas_call(
        _tc_body,
        grid=(C,),
        in_specs=[
            pl.BlockSpec((1, BATCH, ROWS, LANES), lambda c: (c, 0, 0, 0)),
            pl.BlockSpec((1, BATCH, ROWS, LANES), lambda c: (c, 0, 0, 0)),
            pl.BlockSpec((ROWS, LANES), lambda c: (0, 0)),
        ],
        out_specs=pl.BlockSpec(memory_space=pltpu.SMEM),
        out_shape=jax.ShapeDtypeStruct((1, 1), jnp.float32),
        scratch_shapes=[pltpu.SMEM((2,), jnp.float32)],
    )(nh, gh, jnp.asarray(_DV))


def kernel(logits, targets):
    logits_flat = logits.reshape(BATCH, C, NPIX)
    labels_flat = targets.reshape(BATCH, NPIX).astype(jnp.int32)
    nh, gh = _sc_histograms(logits_flat, labels_flat)
    out = _tc_reduce(nh.reshape(C, BATCH, ROWS, LANES),
                     gh.reshape(C, BATCH, ROWS, LANES))
    return out.reshape(())


# speedup vs baseline: 72.8451x; 1.2176x over previous
import functools

import jax
import jax.numpy as jnp
import numpy as np
from jax import lax
from jax.experimental import pallas as pl
from jax.experimental.pallas import tpu as pltpu
from jax.experimental.pallas import tpu_sc as plsc

NB = 32768
SHIFT = 16
NPIX = 512 * 512
C = 19
BATCH = 4
UNITS = C * BATCH
NWORKERS = 32
CHUNK = 8192
NCHUNK = NPIX // CHUNK
UNROLL = 8
ROWS = 256
LANES = 128


def _dv_table() -> np.ndarray:
    b = np.arange(NB, dtype=np.uint64)
    pat = np.minimum((b << SHIFT) + (1 << (SHIFT - 1)), 0x7F7FFFFF)
    v = pat.astype(np.uint32).view(np.float32).astype(np.float64)
    dv = np.diff(np.concatenate([[0.0], v]))
    return dv.reshape(ROWS, LANES).astype(np.float32)


_DV = _dv_table()


def _sc_hist_body(logits_ref, labels_ref, out_n_ref, out_g_ref,
                  nhist, ghist, logbuf, lblbuf, seml, semt):
    wid = lax.axis_index("s") * 2 + lax.axis_index("c")

    def run_unit(u):
        c = u // BATCH
        b = u % BATCH

        def zero_body(i, _):
            z = jnp.zeros((16,), jnp.float32)
            nhist[pl.ds(i * 16, 16)] = z
            ghist[pl.ds(i * 16, 16)] = z
            return 0

        lax.fori_loop(0, NB // 16, zero_body, 0)

        def copies(k):
            s = k % 2
            off = k * CHUNK
            return (
                pltpu.make_async_copy(
                    logits_ref.at[b, c, pl.ds(off, CHUNK)], logbuf.at[s],
                    seml.at[s]),
                pltpu.make_async_copy(
                    labels_ref.at[b, pl.ds(off, CHUNK)], lblbuf.at[s],
                    semt.at[s]),
            )

        def process_vreg(lbl, p):
            fg = lbl == c
            e = jnp.abs(jnp.where(fg, 1.0 - p, p))
            key = lax.bitcast_convert_type(e, jnp.int32) >> SHIFT
            return key, fg

        for cp in copies(0):
            cp.start()

        def chunk_body(k, _):
            s = k % 2

            @pl.when(k + 1 < NCHUNK)
            def _():
                for cp in copies(k + 1):
                    cp.start()

            for cp in copies(k):
                cp.wait()

            def vreg_body(i, _):
                base = i * (16 * UNROLL)
                kfs = []
                for uu in range(UNROLL):
                    lbl = lblbuf[s, pl.ds(base + uu * 16, 16)]
                    p = logbuf[s, pl.ds(base + uu * 16, 16)]
                    kfs.append(process_vreg(lbl, p))
                scans = []
                for key, fg in kfs:
                    scans.append(plsc.scan_count(key))
                    scans.append(plsc.scan_count(key, mask=fg))
                for ui, (key, fg) in enumerate(kfs):
                    cnt_n, last_n = scans[2 * ui]
                    cnt_g, last_g = scans[2 * ui + 1]
                    plsc.addupdate_scatter(nhist, [key],
                                           cnt_n.astype(jnp.float32),
                                           mask=last_n)
                    plsc.addupdate_scatter(ghist, [key],
                                           cnt_g.astype(jnp.float32),
                                           mask=last_g)
                return 0

            lax.fori_loop(0, CHUNK // (16 * UNROLL), vreg_body, 0)
            return 0

        lax.fori_loop(0, NCHUNK, chunk_body, 0)

        pltpu.sync_copy(nhist, out_n_ref.at[c, b])
        pltpu.sync_copy(ghist, out_g_ref.at[c, b])

    for k in range(3):
        u = wid + k * NWORKERS
        if k * NWORKERS + NWORKERS <= UNITS:
            run_unit(u)
        else:
            @pl.when(u < UNITS)
            def _():
                run_unit(u)


def _sc_histograms(logits_flat, labels_flat):
    mesh = plsc.VectorSubcoreMesh(core_axis_name="c", subcore_axis_name="s",
                                  num_cores=2, num_subcores=16)
    kern = pl.kernel(
        _sc_hist_body,
        out_type=(
            jax.ShapeDtypeStruct((C, BATCH, NB), jnp.float32),
            jax.ShapeDtypeStruct((C, BATCH, NB), jnp.float32),
        ),
        mesh=mesh,
        compiler_params=pltpu.CompilerParams(needs_layout_passes=False),
        scratch_types=[
            pltpu.VMEM((NB,), jnp.float32),
            pltpu.VMEM((NB,), jnp.float32),
            pltpu.VMEM((2, CHUNK), jnp.float32),
            pltpu.VMEM((2, CHUNK), jnp.int32),
            pltpu.SemaphoreType.DMA((2,)),
            pltpu.SemaphoreType.DMA((2,)),
        ],
    )
    return kern(logits_flat, labels_flat)


def _tc_body(nh_ref, gh_ref, dv_ref, out_ref, acc_ref):
    c = pl.program_id(0)

    n = jnp.sum(nh_ref[0], axis=0)
    g = jnp.sum(gh_ref[0], axis=0)

    ntot = jnp.sum(n)
    gtot = jnp.sum(g)

    li = lax.broadcasted_iota(jnp.int32, (LANES, LANES), 0)
    lj = lax.broadcasted_iota(jnp.int32, (LANES, LANES), 1)
    lower_incl = (li >= lj).astype(jnp.float32)
    ri = lax.broadcasted_iota(jnp.int32, (ROWS, ROWS), 0)
    rj = lax.broadcasted_iota(jnp.int32, (ROWS, ROWS), 1)
    ustrict = (rj > ri).astype(jnp.float32)

    def suffix_incl(x):
        row_suf = jnp.dot(x, lower_incl, preferred_element_type=jnp.float32,
                          precision=lax.Precision.HIGHEST)
        row_tot = row_suf[:, 0:1]
        row_off = jnp.dot(ustrict, row_tot, preferred_element_type=jnp.float32,
                          precision=lax.Precision.HIGHEST)
        return row_suf + row_off

    n_ge = suffix_incl(n)
    g_ge = suffix_incl(g)
    denom = gtot + n_ge - g_ge
    j = jnp.where(n_ge > 0, n_ge / jnp.maximum(denom, 1e-30), 0.0)
    loss_c = jnp.sum(dv_ref[...] * j)
    present = (gtot > 0).astype(jnp.float32)

    @pl.when(c == 0)
    def _():
        acc_ref[0] = 0.0
        acc_ref[1] = 0.0

    acc_ref[0] += loss_c * present
    acc_ref[1] += present

    @pl.when(c == C - 1)
    def _():
        out_ref[0, 0] = acc_ref[0] / jnp.maximum(acc_ref[1], 1.0)


def _tc_reduce(nh, gh):
    return pl.pallas_call(
        _tc_body,
        grid=(C,),
        in_specs=[
            pl.BlockSpec((1, BATCH, ROWS, LANES), lambda c: (c, 0, 0, 0)),
            pl.BlockSpec((1, BATCH, ROWS, LANES), lambda c: (c, 0, 0, 0)),
            pl.BlockSpec((ROWS, LANES), lambda c: (0, 0)),
        ],
        out_specs=pl.BlockSpec(memory_space=pltpu.SMEM),
        out_shape=jax.ShapeDtypeStruct((1, 1), jnp.float32),
        scratch_shapes=[pltpu.SMEM((2,), jnp.float32)],
    )(nh, gh, jnp.asarray(_DV))


def kernel(logits, targets):
    logits_flat = logits.reshape(BATCH, C, NPIX)
    labels_flat = targets.reshape(BATCH, NPIX).astype(jnp.int32)
    nh, gh = _sc_histograms(logits_flat, labels_flat)
    out = _tc_reduce(nh.reshape(C, BATCH, ROWS, LANES),
                     gh.reshape(C, BATCH, ROWS, LANES))
    return out.reshape(())


# speedup vs baseline: 127.2959x; 1.7475x over previous
import functools

import jax
import jax.numpy as jnp
import numpy as np
from jax import lax
from jax.experimental import pallas as pl
from jax.experimental.pallas import tpu as pltpu
from jax.experimental.pallas import tpu_sc as plsc

NB = 32768
SHIFT = 16
NPIX = 512 * 512
C = 19
BATCH = 4
UNITS = C * BATCH
NWORKERS = 32
CHUNK = 8192
NCHUNK = NPIX // CHUNK
UNROLL = 8
ROWS = 256
LANES = 128


def _dv_table() -> np.ndarray:
    b = np.arange(NB, dtype=np.uint64)
    pat = np.minimum((b << SHIFT) + (1 << (SHIFT - 1)), 0x7F7FFFFF)
    v = pat.astype(np.uint32).view(np.float32).astype(np.float64)
    dv = np.diff(np.concatenate([[0.0], v]))
    return dv.reshape(ROWS, LANES).astype(np.float32)


_DV = _dv_table()


def _sc_hist_body(logits_ref, labels_ref, out_n_ref, out_g_ref,
                  nhist, ghist, logbuf, lblbuf, seml, semt):
    wid = lax.axis_index("s") * 2 + lax.axis_index("c")

    def run_unit(u):
        c = u // BATCH
        b = u % BATCH

        @functools.partial(plsc.parallel_loop, 0, NB // 16)
        def _(i):
            z = jnp.zeros((16,), jnp.int32)
            nhist[pl.ds(i * 16, 16)] = z
            ghist[pl.ds(i * 16, 16)] = z

        def copies(k):
            s = k % 2
            off = k * CHUNK
            return (
                pltpu.make_async_copy(
                    logits_ref.at[b, c, pl.ds(off, CHUNK)], logbuf.at[s],
                    seml.at[s]),
                pltpu.make_async_copy(
                    labels_ref.at[b, pl.ds(off, CHUNK)], lblbuf.at[s],
                    semt.at[s]),
            )

        def process_vreg(lbl, p):
            fg = lbl == c
            e = jnp.abs(jnp.where(fg, 1.0 - p, p))
            key = lax.bitcast_convert_type(e, jnp.int32) >> SHIFT
            return key, fg

        for cp in copies(0):
            cp.start()

        def chunk_body(k, _):
            s = k % 2

            @pl.when(k + 1 < NCHUNK)
            def _():
                for cp in copies(k + 1):
                    cp.start()

            for cp in copies(k):
                cp.wait()

            @functools.partial(plsc.parallel_loop, 0, CHUNK // 16,
                               unroll=UNROLL)
            def _(i):
                lbl = lblbuf[s, pl.ds(i * 16, 16)]
                p = logbuf[s, pl.ds(i * 16, 16)]
                key, fg = process_vreg(lbl, p)
                cnt_n, last_n = plsc.scan_count(key)
                cnt_g, last_g = plsc.scan_count(key, mask=fg)
                plsc.addupdate_scatter(nhist, [key], cnt_n, mask=last_n)
                plsc.addupdate_scatter(ghist, [key], cnt_g, mask=last_g)

            return 0

        lax.fori_loop(0, NCHUNK, chunk_body, 0)

        pltpu.sync_copy(nhist, out_n_ref.at[c, b])
        pltpu.sync_copy(ghist, out_g_ref.at[c, b])

    for k in range(3):
        u = wid + k * NWORKERS
        if k * NWORKERS + NWORKERS <= UNITS:
            run_unit(u)
        else:
            @pl.when(u < UNITS)
            def _():
                run_unit(u)


def _sc_histograms(logits_flat, labels_flat):
    mesh = plsc.VectorSubcoreMesh(core_axis_name="c", subcore_axis_name="s",
                                  num_cores=2, num_subcores=16)
    kern = pl.kernel(
        _sc_hist_body,
        out_type=(
            jax.ShapeDtypeStruct((C, BATCH, NB), jnp.int32),
            jax.ShapeDtypeStruct((C, BATCH, NB), jnp.int32),
        ),
        mesh=mesh,
        compiler_params=pltpu.CompilerParams(needs_layout_passes=False),
        scratch_types=[
            pltpu.VMEM((NB,), jnp.int32),
            pltpu.VMEM((NB,), jnp.int32),
            pltpu.VMEM((2, CHUNK), jnp.float32),
            pltpu.VMEM((2, CHUNK), jnp.int32),
            pltpu.SemaphoreType.DMA((2,)),
            pltpu.SemaphoreType.DMA((2,)),
        ],
    )
    return kern(logits_flat, labels_flat)


def _tc_body(nh_ref, gh_ref, dv_ref, out_ref, acc_ref):
    c = pl.program_id(0)

    n = jnp.sum(nh_ref[0], axis=0).astype(jnp.float32)
    g = jnp.sum(gh_ref[0], axis=0).astype(jnp.float32)

    ntot = jnp.sum(n)
    gtot = jnp.sum(g)

    li = lax.broadcasted_iota(jnp.int32, (LANES, LANES), 0)
    lj = lax.broadcasted_iota(jnp.int32, (LANES, LANES), 1)
    lower_incl = (li >= lj).astype(jnp.float32)
    ri = lax.broadcasted_iota(jnp.int32, (ROWS, ROWS), 0)
    rj = lax.broadcasted_iota(jnp.int32, (ROWS, ROWS), 1)
    ustrict = (rj > ri).astype(jnp.float32)

    def suffix_incl(x):
        row_suf = jnp.dot(x, lower_incl, preferred_element_type=jnp.float32,
                          precision=lax.Precision.HIGHEST)
        row_tot = row_suf[:, 0:1]
        row_off = jnp.dot(ustrict, row_tot, preferred_element_type=jnp.float32,
                          precision=lax.Precision.HIGHEST)
        return row_suf + row_off

    n_ge = suffix_incl(n)
    g_ge = suffix_incl(g)
    denom = gtot + n_ge - g_ge
    j = jnp.where(n_ge > 0, n_ge / jnp.maximum(denom, 1e-30), 0.0)
    loss_c = jnp.sum(dv_ref[...] * j)
    present = (gtot > 0).astype(jnp.float32)

    @pl.when(c == 0)
    def _():
        acc_ref[0] = 0.0
        acc_ref[1] = 0.0

    acc_ref[0] += loss_c * present
    acc_ref[1] += present

    @pl.when(c == C - 1)
    def _():
        out_ref[0, 0] = acc_ref[0] / jnp.maximum(acc_ref[1], 1.0)


def _tc_reduce(nh, gh):
    return pl.pallas_call(
        _tc_body,
        grid=(C,),
        in_specs=[
            pl.BlockSpec((1, BATCH, ROWS, LANES), lambda c: (c, 0, 0, 0)),
            pl.BlockSpec((1, BATCH, ROWS, LANES), lambda c: (c, 0, 0, 0)),
            pl.BlockSpec((ROWS, LANES), lambda c: (0, 0)),
        ],
        out_specs=pl.BlockSpec(memory_space=pltpu.SMEM),
        out_shape=jax.ShapeDtypeStruct((1, 1), jnp.float32),
        scratch_shapes=[pltpu.SMEM((2,), jnp.float32)],
    )(nh, gh, jnp.asarray(_DV))


def kernel(logits, targets):
    logits_flat = logits.reshape(BATCH, C, NPIX)
    labels_flat = targets.reshape(BATCH, NPIX).astype(jnp.int32)
    nh, gh = _sc_histograms(logits_flat, labels_flat)
    out = _tc_reduce(nh.reshape(C, BATCH, ROWS, LANES),
                     gh.reshape(C, BATCH, ROWS, LANES))
    return out.reshape(())
